# Initial kernel scaffold; baseline (speedup 1.0000x reference)
import jax
import jax.numpy as jnp
from jax import lax
from jax.experimental import pallas as pl
from jax.experimental.pallas import tpu as pltpu
from jax.experimental.pallas import tpu_sc as plsc

N = 10000
D = 128
DH = 64
NP = 10240
NS = 16
RPT = NP // NS
CH = 128
E = 320000
NCH32 = -(-E // (32 * CH))
ETOT = 32 * NCH32 * CH
NCH16 = 2 * NCH32
PAD_IDX = NP - 1

_mesh = plsc.VectorSubcoreMesh(core_axis_name="c", subcore_axis_name="s")

_f32 = jnp.float32


def _deg_body(dst_hbm, deg_out, idx_v, ones_v, zbuf, deg_sp):
    c = lax.axis_index("c")
    s = lax.axis_index("s")
    t32 = c * NS + s

    def _fill(i, _):
        ones_v[pl.ds(i * 16, 16)] = jnp.ones((16,), _f32)
        zbuf[pl.ds(i * 16, 16)] = jnp.zeros((16,), _f32)
        return ()

    lax.fori_loop(0, RPT // 16, _fill, ())

    pltpu.sync_copy(dst_hbm.at[t32], idx_v)
    pltpu.sync_copy(zbuf, deg_sp.at[pl.ds(s * RPT, RPT)])
    plsc.subcore_barrier()

    def _chunk(j, _):
        pltpu.sync_copy(ones_v.at[pl.ds(0, CH)], deg_sp.at[idx_v.at[j]],
                        add=True)
        return ()

    lax.fori_loop(0, NCH32, _chunk, ())
    plsc.subcore_barrier()

    @pl.when(s == 0)
    def _():
        pltpu.sync_copy(deg_sp, deg_out.at[c])


_deg_call = pl.kernel(
    _deg_body,
    out_type=jax.ShapeDtypeStruct((2, NP), _f32),
    mesh=_mesh,
    scratch_types=[
        pltpu.VMEM((NCH32, CH), jnp.int32),
        pltpu.VMEM((RPT,), _f32),
        pltpu.VMEM((RPT,), _f32),
        pltpu.VMEM_SHARED((NP,), _f32),
    ],
)


def _hop_body(g_hbm, src_hbm, dst_hbm, a_out,
              sidx, didx, rows0, rows1, zbuf, acc_sp, sem0, sem1):
    c = lax.axis_index("c")
    s = lax.axis_index("s")

    pltpu.sync_copy(src_hbm.at[s], sidx)
    pltpu.sync_copy(dst_hbm.at[s], didx)

    def _zero(i, _):
        for k in range(DH // 16):
            zbuf[i, pl.ds(k * 16, 16)] = jnp.zeros((16,), _f32)
        return ()

    lax.fori_loop(0, CH, _zero, ())
    for q in range(RPT // CH):
        pltpu.sync_copy(zbuf, acc_sp.at[pl.ds(s * RPT + q * CH, CH)])
    plsc.subcore_barrier()

    gsrc = g_hbm.at[c]

    pltpu.async_copy(gsrc.at[sidx.at[0]], rows0, sem0)

    def _pair(jj, _):
        j0 = 2 * jj
        j1 = j0 + 1
        pltpu.make_async_copy(gsrc.at[sidx.at[j0]], rows0, sem0).wait()
        pltpu.async_copy(gsrc.at[sidx.at[j1]], rows1, sem1)
        pltpu.sync_copy(rows0, acc_sp.at[didx.at[j0]], add=True)

        @pl.when(j1 + 1 < NCH16)
        def _():
            pltpu.async_copy(gsrc.at[sidx.at[j1 + 1]], rows0, sem0)

        pltpu.make_async_copy(gsrc.at[sidx.at[j1]], rows1, sem1).wait()
        pltpu.sync_copy(rows1, acc_sp.at[didx.at[j1]], add=True)
        return ()

    lax.fori_loop(0, NCH16 // 2, _pair, ())
    plsc.subcore_barrier()
    pltpu.sync_copy(acc_sp.at[pl.ds(s * RPT, RPT)],
                    a_out.at[c, pl.ds(s * RPT, RPT)])


_hop_call = pl.kernel(
    _hop_body,
    out_type=jax.ShapeDtypeStruct((2, NP, DH), _f32),
    mesh=_mesh,
    scratch_types=[
        pltpu.VMEM((NCH16, CH), jnp.int32),
        pltpu.VMEM((NCH16, CH), jnp.int32),
        pltpu.VMEM((CH, DH), _f32),
        pltpu.VMEM((CH, DH), _f32),
        pltpu.VMEM((CH, DH), _f32),
        pltpu.VMEM_SHARED((NP, DH), _f32),
        pltpu.SemaphoreType.DMA,
        pltpu.SemaphoreType.DMA,
    ],
)


R = 512


def _prep_body(deg_ref, feat_ref, norm_ref, inv_ref, g0_ref):
    d = jnp.maximum(deg_ref[0] + deg_ref[1], 1.0)
    nr = lax.rsqrt(d)
    norm_ref[...] = nr
    inv_ref[...] = nr * nr
    g0_ref[...] = feat_ref[...] * nr[None]


def _tc_prep(deg2, feat2):
    return pl.pallas_call(
        _prep_body,
        grid=(NP // R,),
        in_specs=[
            pl.BlockSpec((2, R, 1), lambda r: (0, r, 0)),
            pl.BlockSpec((2, R, DH), lambda r: (0, r, 0)),
        ],
        out_specs=[
            pl.BlockSpec((R, 1), lambda r: (r, 0)),
            pl.BlockSpec((R, 1), lambda r: (r, 0)),
            pl.BlockSpec((2, R, DH), lambda r: (0, r, 0)),
        ],
        out_shape=[
            jax.ShapeDtypeStruct((NP, 1), _f32),
            jax.ShapeDtypeStruct((NP, 1), _f32),
            jax.ShapeDtypeStruct((2, NP, DH), _f32),
        ],
    )(deg2, feat2)


def _scale_body(inv_ref, a_ref, g_ref):
    g_ref[...] = a_ref[...] * inv_ref[...][None]


def _tc_scale(inv, a1):
    return pl.pallas_call(
        _scale_body,
        grid=(NP // R,),
        in_specs=[
            pl.BlockSpec((R, 1), lambda r: (r, 0)),
            pl.BlockSpec((2, R, DH), lambda r: (0, r, 0)),
        ],
        out_specs=pl.BlockSpec((2, R, DH), lambda r: (0, r, 0)),
        out_shape=jax.ShapeDtypeStruct((2, NP, DH), _f32),
    )(inv, a1)


def _final_body(norm_ref, a2_ref, feat_ref, wfc_ref, wres_ref, b_ref,
                out_ref):
    nr = norm_ref[...]
    h_lo = a2_ref[0] * nr
    h_hi = a2_ref[1] * nr
    acc = jnp.dot(h_lo, wfc_ref[pl.ds(0, DH), :],
                  preferred_element_type=_f32)
    acc += jnp.dot(h_hi, wfc_ref[pl.ds(DH, DH), :],
                   preferred_element_type=_f32)
    acc += jnp.dot(feat_ref[...], wres_ref[...],
                   preferred_element_type=_f32)
    out_ref[...] = acc + b_ref[...]


def _tc_final(norm, a2, feat_pad, W_fc, W_res, bias):
    return pl.pallas_call(
        _final_body,
        grid=(NP // R,),
        in_specs=[
            pl.BlockSpec((R, 1), lambda r: (r, 0)),
            pl.BlockSpec((2, R, DH), lambda r: (0, r, 0)),
            pl.BlockSpec((R, D), lambda r: (r, 0)),
            pl.BlockSpec((D, D), lambda r: (0, 0)),
            pl.BlockSpec((D, D), lambda r: (0, 0)),
            pl.BlockSpec((1, D), lambda r: (0, 0)),
        ],
        out_specs=pl.BlockSpec((R, D), lambda r: (r, 0)),
        out_shape=jax.ShapeDtypeStruct((NP, D), _f32),
    )(norm, a2, feat_pad, W_fc, W_res, bias)


def kernel(feat, edge_index, W_fc, b_fc, W_res, b_res):
    src = edge_index[0]
    dst = edge_index[1]
    pad = jnp.full((ETOT - E,), PAD_IDX, jnp.int32)
    src_p = jnp.concatenate([src, pad])
    dst_p = jnp.concatenate([dst, pad])
    dst32 = dst_p.reshape(32, NCH32, CH)
    src16 = src_p.reshape(NS, NCH16, CH)
    dst16 = dst_p.reshape(NS, NCH16, CH)

    feat_pad = jnp.pad(feat, ((0, NP - N), (0, 0)))
    feat2 = feat_pad.reshape(NP, 2, DH).transpose(1, 0, 2)

    deg2 = _deg_call(dst32)
    norm, inv, g0 = _tc_prep(deg2[..., None], feat2)
    a1 = _hop_call(g0, src16, dst16)
    g1 = _tc_scale(inv, a1)
    a2 = _hop_call(g1, src16, dst16)
    bias = (b_fc + b_res)[None, :]
    out_pad = _tc_final(norm, a2, feat_pad, W_fc, W_res, bias)
    return out_pad[:N]


# speedup vs baseline: 5.8975x; 5.8975x over previous
import jax
import jax.numpy as jnp
from jax import lax
from jax.experimental import pallas as pl
from jax.experimental.pallas import tpu as pltpu
from jax.experimental.pallas import tpu_sc as plsc

N = 10000
D = 128
DH = 64
NP = 10240
NS = 16
RPT = NP // NS
CH = 128
E = 320000
NCH32 = -(-E // (32 * CH))
ETOT = 32 * NCH32 * CH
NCH = 2 * NCH32
PAD_IDX = NP - 1

_mesh = plsc.VectorSubcoreMesh(core_axis_name="c", subcore_axis_name="s")

_f32 = jnp.float32

_sc_params = pltpu.CompilerParams(use_tc_tiling_on_sc=False)


def _deg_body(dst_hbm, deg_out, idx_v, ones_v, zbuf, deg_sp):
    c = lax.axis_index("c")
    s = lax.axis_index("s")

    def _fill(i, _):
        ones_v[pl.ds(i * 16, 16)] = jnp.ones((16,), _f32)
        zbuf[pl.ds(i * 16, 16)] = jnp.zeros((16,), _f32)
        return ()

    lax.fori_loop(0, RPT // 16, _fill, ())

    pltpu.sync_copy(dst_hbm.at[c, s], idx_v)
    pltpu.sync_copy(zbuf, deg_sp.at[pl.ds(s * RPT, RPT)])
    plsc.subcore_barrier()

    def _chunk(j, _):
        pltpu.sync_copy(ones_v.at[pl.ds(0, CH)], deg_sp.at[idx_v.at[j]],
                        add=True)
        return ()

    lax.fori_loop(0, NCH32, _chunk, ())
    plsc.subcore_barrier()

    @pl.when(s == 0)
    def _():
        pltpu.sync_copy(deg_sp, deg_out.at[c])


_deg_call = pl.kernel(
    _deg_body,
    out_type=jax.ShapeDtypeStruct((2, NP), _f32),
    mesh=_mesh,
    scratch_types=[
        pltpu.VMEM((NCH32, CH), jnp.int32),
        pltpu.VMEM((RPT,), _f32),
        pltpu.VMEM((RPT,), _f32),
        pltpu.VMEM_SHARED((NP,), _f32),
    ],
    compiler_params=_sc_params,
)


def _hop_body(g_hbm, src_hbm, dst_hbm, a_out,
              sidx, didx, rows0, rows1, zbuf, acc_sp, sem0, sem1):
    c = lax.axis_index("c")
    s = lax.axis_index("s")

    pltpu.sync_copy(src_hbm.at[s], sidx)
    pltpu.sync_copy(dst_hbm.at[s], didx)

    def _zero(i, _):
        for k in range(DH // 16):
            zbuf[i, pl.ds(k * 16, 16)] = jnp.zeros((16,), _f32)
        return ()

    lax.fori_loop(0, CH, _zero, ())
    for q in range(RPT // CH):
        pltpu.sync_copy(zbuf, acc_sp.at[pl.ds(s * RPT + q * CH, CH)])
    plsc.subcore_barrier()

    gsrc = g_hbm.at[c]

    pltpu.async_copy(gsrc.at[sidx.at[0]], rows0, sem0)

    def _pair(jj, _):
        j0 = 2 * jj
        j1 = j0 + 1
        pltpu.make_async_copy(gsrc.at[sidx.at[j0]], rows0, sem0).wait()
        pltpu.async_copy(gsrc.at[sidx.at[j1]], rows1, sem1)
        pltpu.sync_copy(rows0, acc_sp.at[didx.at[j0]], add=True)

        @pl.when(j1 + 1 < NCH)
        def _():
            pltpu.async_copy(gsrc.at[sidx.at[j1 + 1]], rows0, sem0)

        pltpu.make_async_copy(gsrc.at[sidx.at[j1]], rows1, sem1).wait()
        pltpu.sync_copy(rows1, acc_sp.at[didx.at[j1]], add=True)
        return ()

    lax.fori_loop(0, NCH // 2, _pair, ())
    plsc.subcore_barrier()
    pltpu.sync_copy(acc_sp.at[pl.ds(s * RPT, RPT)],
                    a_out.at[c, pl.ds(s * RPT, RPT)])


_hop_call = pl.kernel(
    _hop_body,
    out_type=jax.ShapeDtypeStruct((2, NP, DH), _f32),
    mesh=_mesh,
    scratch_types=[
        pltpu.VMEM((NCH, CH), jnp.int32),
        pltpu.VMEM((NCH, CH), jnp.int32),
        pltpu.VMEM((CH, DH), _f32),
        pltpu.VMEM((CH, DH), _f32),
        pltpu.VMEM((CH, DH), _f32),
        pltpu.VMEM_SHARED((NP, DH), _f32),
        pltpu.SemaphoreType.DMA,
        pltpu.SemaphoreType.DMA,
    ],
    compiler_params=_sc_params,
)


R = 512


def _prep_body(deg_ref, feat_ref, norm_ref, inv_ref, g0_ref):
    d = jnp.maximum(deg_ref[0] + deg_ref[1], 1.0)
    nr = lax.rsqrt(d)
    norm_ref[...] = nr
    inv_ref[...] = nr * nr
    g0_ref[...] = feat_ref[...] * nr[None]


def _tc_prep(deg2, feat2):
    return pl.pallas_call(
        _prep_body,
        grid=(NP // R,),
        in_specs=[
            pl.BlockSpec((2, R, 1), lambda r: (0, r, 0)),
            pl.BlockSpec((2, R, DH), lambda r: (0, r, 0)),
        ],
        out_specs=[
            pl.BlockSpec((R, 1), lambda r: (r, 0)),
            pl.BlockSpec((R, 1), lambda r: (r, 0)),
            pl.BlockSpec((2, R, DH), lambda r: (0, r, 0)),
        ],
        out_shape=[
            jax.ShapeDtypeStruct((NP, 1), _f32),
            jax.ShapeDtypeStruct((NP, 1), _f32),
            jax.ShapeDtypeStruct((2, NP, DH), _f32),
        ],
    )(deg2, feat2)


def _scale_body(inv_ref, a_ref, g_ref):
    g_ref[...] = a_ref[...] * inv_ref[...][None]


def _tc_scale(inv, a1):
    return pl.pallas_call(
        _scale_body,
        grid=(NP // R,),
        in_specs=[
            pl.BlockSpec((R, 1), lambda r: (r, 0)),
            pl.BlockSpec((2, R, DH), lambda r: (0, r, 0)),
        ],
        out_specs=pl.BlockSpec((2, R, DH), lambda r: (0, r, 0)),
        out_shape=jax.ShapeDtypeStruct((2, NP, DH), _f32),
    )(inv, a1)


def _final_body(norm_ref, a2_ref, feat_ref, wfc_ref, wres_ref, b_ref,
                out_ref):
    nr = norm_ref[...]
    h_lo = a2_ref[0] * nr
    h_hi = a2_ref[1] * nr
    acc = jnp.dot(h_lo, wfc_ref[pl.ds(0, DH), :],
                  preferred_element_type=_f32)
    acc += jnp.dot(h_hi, wfc_ref[pl.ds(DH, DH), :],
                   preferred_element_type=_f32)
    acc += jnp.dot(feat_ref[...], wres_ref[...],
                   preferred_element_type=_f32)
    out_ref[...] = acc + b_ref[...]


def _tc_final(norm, a2, feat_pad, W_fc, W_res, bias):
    return pl.pallas_call(
        _final_body,
        grid=(NP // R,),
        in_specs=[
            pl.BlockSpec((R, 1), lambda r: (r, 0)),
            pl.BlockSpec((2, R, DH), lambda r: (0, r, 0)),
            pl.BlockSpec((R, D), lambda r: (r, 0)),
            pl.BlockSpec((D, D), lambda r: (0, 0)),
            pl.BlockSpec((D, D), lambda r: (0, 0)),
            pl.BlockSpec((1, D), lambda r: (0, 0)),
        ],
        out_specs=pl.BlockSpec((R, D), lambda r: (r, 0)),
        out_shape=jax.ShapeDtypeStruct((NP, D), _f32),
    )(norm, a2, feat_pad, W_fc, W_res, bias)


def kernel(feat, edge_index, W_fc, b_fc, W_res, b_res):
    src = edge_index[0]
    dst = edge_index[1]
    pad = jnp.full((ETOT - E,), PAD_IDX, jnp.int32)
    src16 = jnp.concatenate([src, pad]).reshape(NS, NCH, CH)
    dst16 = jnp.concatenate([dst, pad]).reshape(NS, NCH, CH)
    dst32 = dst16.reshape(2, NS, NCH32, CH)

    feat_pad = jnp.pad(feat, ((0, NP - N), (0, 0)))
    feat2 = feat_pad.reshape(NP, 2, DH).transpose(1, 0, 2)

    deg2 = _deg_call(dst32)
    norm, inv, g0 = _tc_prep(deg2[..., None], feat2)
    a1 = _hop_call(g0, src16, dst16)
    g1 = _tc_scale(inv, a1)
    a2 = _hop_call(g1, src16, dst16)
    bias = (b_fc + b_res)[None, :]
    out_pad = _tc_final(norm, a2, feat_pad, W_fc, W_res, bias)
    return out_pad[:N]
